# baseline (speedup 1.0000x reference)
import functools

import jax
import jax.numpy as jnp
from jax import lax
from jax.experimental import pallas as pl
from jax.experimental.pallas import tpu as pltpu
from jax.experimental.pallas import tpu_sc as plsc

N_NODES = 1000000
DIM = 64
BATCH = 16384
_PAIR = 2 * DIM

_info = plsc.get_sparse_core_info()
_NC, _NS = _info.num_cores, _info.num_subcores
_NW = _NC * _NS
_B_PER_W = BATCH // _NW
_VLEN = 16


@functools.partial(
    pl.kernel,
    mesh=plsc.VectorSubcoreMesh(core_axis_name="c", subcore_axis_name="s"),
    out_type=jax.ShapeDtypeStruct((BATCH * DIM,), jnp.float32),
    scratch_types=[
        pltpu.VMEM((_B_PER_W,), jnp.int32),
        pltpu.VMEM((_B_PER_W,), jnp.int32),
        pltpu.VMEM((_B_PER_W, _PAIR), jnp.float32),
        pltpu.VMEM((_B_PER_W * DIM,), jnp.float32),
        pltpu.SemaphoreType.DMA,
    ],
)
def _gather_kernel(pairs_hbm, idx_hbm, out_hbm, idx_v, row_v, rows_v, outb_v, sem):
    wid = lax.axis_index("s") * _NC + lax.axis_index("c")
    base = wid * _B_PER_W
    pltpu.sync_copy(idx_hbm.at[pl.ds(base, _B_PER_W)], idx_v)

    def to_rows(t, carry):
        v = idx_v[pl.ds(t * _VLEN, _VLEN)]
        row_v[pl.ds(t * _VLEN, _VLEN)] = lax.shift_right_logical(v, 1)
        return carry

    lax.fori_loop(0, _B_PER_W // _VLEN, to_rows, None, unroll=False)

    pltpu.async_copy(pairs_hbm.at[row_v], rows_v, sem).wait()

    def select(t, carry):
        odd = jnp.bitwise_and(idx_v[pl.ds(t * _VLEN, _VLEN)], 1)
        for b in range(_VLEN):
            k = t * _VLEN + b
            take_hi = odd[b] == 1
            for q in range(DIM // _VLEN):
                lo = rows_v[k, pl.ds(q * _VLEN, _VLEN)]
                hi = rows_v[k, pl.ds(DIM + q * _VLEN, _VLEN)]
                outb_v[pl.ds(k * DIM + q * _VLEN, _VLEN)] = jnp.where(
                    take_hi, hi, lo
                )
        return carry

    lax.fori_loop(0, _B_PER_W // _VLEN, select, None, unroll=False)
    pltpu.sync_copy(outb_v, out_hbm.at[pl.ds(base * DIM, _B_PER_W * DIM)])


@jax.jit
def kernel(data, ivectors):
    one = data[0].astype(jnp.float32) * jnp.float32(1e-38) + jnp.float32(1.0)
    pairs = (ivectors * one).reshape(N_NODES // 2, _PAIR)
    flat = _gather_kernel(pairs, data.astype(jnp.int32))
    return flat.reshape(BATCH, DIM)


# speedup vs baseline: 2.1453x; 2.1453x over previous
import functools

import jax
import jax.numpy as jnp
from jax import lax
from jax.experimental import pallas as pl
from jax.experimental.pallas import tpu as pltpu
from jax.experimental.pallas import tpu_sc as plsc

N_NODES = 1000000
DIM = 64
BATCH = 16384
_H = 512000
_PAIR = 2 * DIM

_info = plsc.get_sparse_core_info()
_NC, _NS = _info.num_cores, _info.num_subcores
_NW = _NC * _NS
_B_PER_W = BATCH // _NW
_VLEN = 16


@functools.partial(
    pl.kernel,
    mesh=plsc.VectorSubcoreMesh(core_axis_name="c", subcore_axis_name="s"),
    out_type=jax.ShapeDtypeStruct((BATCH * DIM,), jnp.float32),
    scratch_types=[
        pltpu.VMEM((_B_PER_W,), jnp.int32),
        pltpu.VMEM((_B_PER_W,), jnp.int32),
        pltpu.VMEM((_B_PER_W, _PAIR), jnp.float32),
        pltpu.VMEM((_B_PER_W * DIM,), jnp.float32),
        pltpu.SemaphoreType.DMA,
    ],
)
def _gather_kernel(pairs_hbm, idx_hbm, out_hbm, idx_v, row_v, rows_v, outb_v, sem):
    wid = lax.axis_index("s") * _NC + lax.axis_index("c")
    base = wid * _B_PER_W
    pltpu.sync_copy(idx_hbm.at[pl.ds(base, _B_PER_W)], idx_v)

    def to_rows(t, carry):
        v = idx_v[pl.ds(t * _VLEN, _VLEN)]
        row_v[pl.ds(t * _VLEN, _VLEN)] = jnp.where(v >= _H, v - _H, v)
        return carry

    lax.fori_loop(0, _B_PER_W // _VLEN, to_rows, None, unroll=False)

    pltpu.async_copy(pairs_hbm.at[row_v], rows_v, sem).wait()

    def select(t, carry):
        v = idx_v[pl.ds(t * _VLEN, _VLEN)]
        for b in range(_VLEN):
            k = t * _VLEN + b
            take_hi = v[b] >= _H
            for q in range(DIM // _VLEN):
                lo = rows_v[k, pl.ds(q * _VLEN, _VLEN)]
                hi = rows_v[k, pl.ds(DIM + q * _VLEN, _VLEN)]
                outb_v[pl.ds(k * DIM + q * _VLEN, _VLEN)] = jnp.where(
                    take_hi, hi, lo
                )
        return carry

    lax.fori_loop(0, _B_PER_W // _VLEN, select, None, unroll=False)
    pltpu.sync_copy(outb_v, out_hbm.at[pl.ds(base * DIM, _B_PER_W * DIM)])


_BN = 2048
_GRID = _H // _BN
_MAXB = (N_NODES - 1) // _BN


def _relayout_body(lo_ref, hi_ref, out_ref):
    lo = lo_ref[...]
    hi = hi_ref[...]
    out_ref[...] = jnp.concatenate([lo.T, hi.T], axis=1)


_relayout = pl.pallas_call(
    _relayout_body,
    grid=(_GRID,),
    in_specs=[
        pl.BlockSpec((DIM, _BN), lambda i: (0, i)),
        pl.BlockSpec((DIM, _BN), lambda i: (0, jnp.minimum(i + _GRID, _MAXB))),
    ],
    out_specs=pl.BlockSpec((_BN, _PAIR), lambda i: (i, 0)),
    out_shape=jax.ShapeDtypeStruct((_H, _PAIR), jnp.float32),
)


@jax.jit
def kernel(data, ivectors):
    tabt = ivectors.T
    pairs = _relayout(tabt, tabt)
    flat = _gather_kernel(pairs, data.astype(jnp.int32))
    return flat.reshape(BATCH, DIM)


# speedup vs baseline: 2.6387x; 1.2300x over previous
import functools

import jax
import jax.numpy as jnp
from jax import lax
from jax.experimental import pallas as pl
from jax.experimental.pallas import tpu as pltpu
from jax.experimental.pallas import tpu_sc as plsc

N_NODES = 1000000
DIM = 64
BATCH = 16384
_H = 512000
_PAIR = 2 * DIM

_info = plsc.get_sparse_core_info()
_NC, _NS = _info.num_cores, _info.num_subcores
_NW = _NC * _NS
_B_PER_W = BATCH // _NW
_VLEN = 16


@functools.partial(
    pl.kernel,
    mesh=plsc.VectorSubcoreMesh(core_axis_name="c", subcore_axis_name="s"),
    out_type=jax.ShapeDtypeStruct((BATCH * DIM,), jnp.float32),
    scratch_types=[
        pltpu.VMEM((_B_PER_W,), jnp.int32),
        pltpu.VMEM((_B_PER_W,), jnp.int32),
        pltpu.VMEM((_B_PER_W, _PAIR), jnp.float32),
        pltpu.VMEM((_B_PER_W * DIM,), jnp.float32),
        pltpu.SemaphoreType.DMA,
    ],
)
def _gather_kernel(pairs_hbm, idx_hbm, out_hbm, idx_v, row_v, rows_v, outb_v, sem):
    wid = lax.axis_index("s") * _NC + lax.axis_index("c")
    base = wid * _B_PER_W
    pltpu.sync_copy(idx_hbm.at[pl.ds(base, _B_PER_W)], idx_v)

    def to_rows(t, carry):
        v = idx_v[pl.ds(t * _VLEN, _VLEN)]
        row_v[pl.ds(t * _VLEN, _VLEN)] = jnp.where(v >= _H, v - _H, v)
        return carry

    lax.fori_loop(0, _B_PER_W // _VLEN, to_rows, None, unroll=False)

    pltpu.async_copy(pairs_hbm.at[row_v], rows_v, sem).wait()

    def select(t, carry):
        v = idx_v[pl.ds(t * _VLEN, _VLEN)]
        for b in range(_VLEN):
            k = t * _VLEN + b
            take_hi = v[b] >= _H
            for q in range(DIM // _VLEN):
                lo = rows_v[k, pl.ds(q * _VLEN, _VLEN)]
                hi = rows_v[k, pl.ds(DIM + q * _VLEN, _VLEN)]
                outb_v[pl.ds(k * DIM + q * _VLEN, _VLEN)] = jnp.where(
                    take_hi, hi, lo
                )
        return carry

    lax.fori_loop(0, _B_PER_W // _VLEN, select, None, unroll=False)
    pltpu.sync_copy(outb_v, out_hbm.at[pl.ds(base * DIM, _B_PER_W * DIM)])


_BN = 4096
_GRID = _H // _BN
_MAXB = (N_NODES - 1) // _BN


def _relayout_body(lo_ref, hi_ref, out_ref):
    lo = lo_ref[...]
    hi = hi_ref[...]
    out_ref[...] = jnp.concatenate([lo.T, hi.T], axis=1)


_relayout = pl.pallas_call(
    _relayout_body,
    grid=(_GRID,),
    in_specs=[
        pl.BlockSpec((DIM, _BN), lambda i: (0, i)),
        pl.BlockSpec((DIM, _BN), lambda i: (0, jnp.minimum(i + _GRID, _MAXB))),
    ],
    out_specs=pl.BlockSpec((_BN, _PAIR), lambda i: (i, 0)),
    out_shape=jax.ShapeDtypeStruct((_H, _PAIR), jnp.float32),
)


@jax.jit
def kernel(data, ivectors):
    tabt = ivectors.T
    pairs = _relayout(tabt, tabt)
    flat = _gather_kernel(pairs, data.astype(jnp.int32))
    return flat.reshape(BATCH, DIM)


# speedup vs baseline: 3.1284x; 1.1856x over previous
import functools

import jax
import jax.numpy as jnp
from jax import lax
from jax.experimental import pallas as pl
from jax.experimental.pallas import tpu as pltpu
from jax.experimental.pallas import tpu_sc as plsc

N_NODES = 1000000
DIM = 64
BATCH = 16384
_H = 512000
_PAIR = 2 * DIM

_info = plsc.get_sparse_core_info()
_NC, _NS = _info.num_cores, _info.num_subcores
_NW = _NC * _NS
_B_PER_W = BATCH // _NW
_VLEN = 16


@functools.partial(
    pl.kernel,
    mesh=plsc.VectorSubcoreMesh(core_axis_name="c", subcore_axis_name="s"),
    out_type=jax.ShapeDtypeStruct((BATCH * DIM,), jnp.float32),
    scratch_types=[
        pltpu.VMEM((_B_PER_W,), jnp.int32),
        pltpu.VMEM((_B_PER_W,), jnp.int32),
        pltpu.VMEM((_B_PER_W, _PAIR), jnp.float32),
        pltpu.VMEM((_B_PER_W * DIM,), jnp.float32),
        pltpu.SemaphoreType.DMA,
    ],
)
def _gather_kernel(pairs_hbm, idx_hbm, out_hbm, idx_v, row_v, rows_v, outb_v, sem):
    wid = lax.axis_index("s") * _NC + lax.axis_index("c")
    base = wid * _B_PER_W
    pltpu.sync_copy(idx_hbm.at[pl.ds(base, _B_PER_W)], idx_v)

    def to_rows(t, carry):
        v = idx_v[pl.ds(t * _VLEN, _VLEN)]
        row_v[pl.ds(t * _VLEN, _VLEN)] = jnp.where(v >= _H, v - _H, v)
        return carry

    lax.fori_loop(0, _B_PER_W // _VLEN, to_rows, None, unroll=False)

    pltpu.async_copy(pairs_hbm.at[row_v], rows_v, sem).wait()

    def select(t, carry):
        v = idx_v[pl.ds(t * _VLEN, _VLEN)]
        for b in range(_VLEN):
            k = t * _VLEN + b
            take_hi = v[b] >= _H
            for q in range(DIM // _VLEN):
                lo = rows_v[k, pl.ds(q * _VLEN, _VLEN)]
                hi = rows_v[k, pl.ds(DIM + q * _VLEN, _VLEN)]
                outb_v[pl.ds(k * DIM + q * _VLEN, _VLEN)] = jnp.where(
                    take_hi, hi, lo
                )
        return carry

    lax.fori_loop(0, _B_PER_W // _VLEN, select, None, unroll=False)
    pltpu.sync_copy(outb_v, out_hbm.at[pl.ds(base * DIM, _B_PER_W * DIM)])


_BN = 16000
_GRID = _H // _BN
_MAXB = (N_NODES - 1) // _BN


def _relayout_body(lo_ref, hi_ref, out_ref):
    lo = lo_ref[...]
    hi = hi_ref[...]
    out_ref[...] = jnp.concatenate([lo.T, hi.T], axis=1)


_relayout = pl.pallas_call(
    _relayout_body,
    grid=(_GRID,),
    in_specs=[
        pl.BlockSpec((DIM, _BN), lambda i: (0, i)),
        pl.BlockSpec((DIM, _BN), lambda i: (0, jnp.minimum(i + _GRID, _MAXB))),
    ],
    out_specs=pl.BlockSpec((_BN, _PAIR), lambda i: (i, 0)),
    out_shape=jax.ShapeDtypeStruct((_H, _PAIR), jnp.float32),
)


@jax.jit
def kernel(data, ivectors):
    tabt = ivectors.T
    pairs = _relayout(tabt, tabt)
    flat = _gather_kernel(pairs, data.astype(jnp.int32))
    return flat.reshape(BATCH, DIM)
